# baseline (speedup 1.0000x reference)
import jax
import jax.numpy as jnp
from jax.experimental import pallas as pl
from jax.experimental.pallas import tpu as pltpu

_R = 4096


def _bf(a):
    return a.astype(jnp.bfloat16)


def _vq_kernel(x_ref, w_enc_ref, b_enc_ref, cb_ref, w_dec_ref,
               b_dec_ref, out_ref, cbb_ref, cn_ref, dec_ref):
    K = cb_ref.shape[0]

    @pl.when(pl.program_id(0) == 0)
    def _init():
        cb = cb_ref[...]
        cbb_ref[...] = _bf(cb)
        cn_ref[...] = jnp.sum(cb * cb, axis=1, keepdims=True)
        wd = _bf(w_dec_ref[...]).astype(jnp.float32)
        dec_ref[...] = jnp.sum(
            _bf(cb).astype(jnp.float32) * wd, axis=1, keepdims=True)

    xb = _bf(x_ref[0])
    flat_t = jax.lax.dot_general(
        _bf(w_enc_ref[...]), xb, (((1,), (1,)), ((), ())),
        preferred_element_type=jnp.float32,
    ) + b_enc_ref[...]
    s = jax.lax.dot_general(
        cbb_ref[...], _bf(flat_t), (((1,), (0,)), ((), ())),
        preferred_element_type=jnp.float32,
    )
    d2 = cn_ref[...] - 2.0 * s
    dec = dec_ref[...]
    k = K
    while k > 1:
        h = k // 2
        mask = d2[h:] < d2[:h]
        d2 = jnp.where(mask, d2[h:], d2[:h])
        dec = jnp.where(mask, dec[h:], dec[:h])
        k = h
    out_ref[0] = dec + b_dec_ref[0, 0]


def _run(x3, w_enc, b_enc_c, cb, w_dec_r, b_dec_r):
    G = x3.shape[0]
    Kc, D = cb.shape
    return pl.pallas_call(
        _vq_kernel,
        grid=(G,),
        in_specs=[
            pl.BlockSpec((1, _R, 2), lambda i: (i, 0, 0)),
            pl.BlockSpec((D, 2), lambda i: (0, 0)),
            pl.BlockSpec((D, 1), lambda i: (0, 0)),
            pl.BlockSpec((Kc, D), lambda i: (0, 0)),
            pl.BlockSpec((1, D), lambda i: (0, 0)),
            pl.BlockSpec((1, 1), lambda i: (0, 0)),
        ],
        out_specs=pl.BlockSpec((1, 1, _R), lambda i: (i, 0, 0)),
        out_shape=jax.ShapeDtypeStruct((G, 1, _R), jnp.float32),
        scratch_shapes=[
            pltpu.VMEM((Kc, D), jnp.bfloat16),
            pltpu.VMEM((Kc, 1), jnp.float32),
            pltpu.VMEM((Kc, 1), jnp.float32),
        ],
    )(x3, w_enc, b_enc_c, cb, w_dec_r, b_dec_r)


def kernel(x, W_enc, b_enc, codebook, W_dec, b_dec):
    T, B, _ = x.shape
    Kc, D = codebook.shape
    n = T * B
    G = n // _R
    out = _run(
        x.reshape(G, _R, 2),
        W_enc.T,
        b_enc.reshape(D, 1),
        codebook,
        W_dec.reshape(1, D),
        b_dec.reshape(1, 1),
    )
    return out.reshape(T, B, 1)


# speedup vs baseline: 1.7663x; 1.7663x over previous
import jax
import jax.numpy as jnp
from jax.experimental import pallas as pl
from jax.experimental.pallas import tpu as pltpu

_R = 8192


def _bf(a):
    return a.astype(jnp.bfloat16)


def _vq_kernel(x0_ref, x1_ref, w_enc_ref, b_enc_ref, cb_ref, w_dec_ref,
               b_dec_ref, out_ref, cbb_ref, cn_ref, dec_ref):
    K = cb_ref.shape[0]

    @pl.when(pl.program_id(0) == 0)
    def _init():
        cb = cb_ref[...]
        cbb_ref[...] = _bf(cb)
        cn_ref[...] = jnp.sum(cb * cb, axis=1, keepdims=True)
        wd = _bf(w_dec_ref[...]).astype(jnp.float32)
        dec_ref[...] = jnp.sum(
            _bf(cb).astype(jnp.float32) * wd, axis=1, keepdims=True)

    x0 = _bf(x0_ref[0]).astype(jnp.float32)
    x1 = _bf(x1_ref[0]).astype(jnp.float32)
    w0 = _bf(w_enc_ref[:, 0:1]).astype(jnp.float32)
    w1 = _bf(w_enc_ref[:, 1:2]).astype(jnp.float32)
    flat_t = (w0 * x0 + w1 * x1) + b_enc_ref[...]
    s = jax.lax.dot_general(
        cbb_ref[...], _bf(flat_t), (((1,), (0,)), ((), ())),
        preferred_element_type=jnp.float32,
    )
    d2 = cn_ref[...] - 2.0 * s
    dec = dec_ref[...]
    k = K
    while k > 1:
        h = k // 2
        mask = d2[h:] < d2[:h]
        d2 = jnp.where(mask, d2[h:], d2[:h])
        dec = jnp.where(mask, dec[h:], dec[:h])
        k = h
    out_ref[0] = dec + b_dec_ref[0, 0]


def _run(x0, x1, w_enc, b_enc_c, cb, w_dec_r, b_dec_r):
    G = x0.shape[0]
    Kc, D = cb.shape
    return pl.pallas_call(
        _vq_kernel,
        grid=(G,),
        in_specs=[
            pl.BlockSpec((1, 1, _R), lambda i: (i, 0, 0)),
            pl.BlockSpec((1, 1, _R), lambda i: (i, 0, 0)),
            pl.BlockSpec((D, 2), lambda i: (0, 0)),
            pl.BlockSpec((D, 1), lambda i: (0, 0)),
            pl.BlockSpec((Kc, D), lambda i: (0, 0)),
            pl.BlockSpec((1, D), lambda i: (0, 0)),
            pl.BlockSpec((1, 1), lambda i: (0, 0)),
        ],
        out_specs=pl.BlockSpec((1, 1, _R), lambda i: (i, 0, 0)),
        out_shape=jax.ShapeDtypeStruct((G, 1, _R), jnp.float32),
        scratch_shapes=[
            pltpu.VMEM((Kc, D), jnp.bfloat16),
            pltpu.VMEM((Kc, 1), jnp.float32),
            pltpu.VMEM((Kc, 1), jnp.float32),
        ],
    )(x0, x1, w_enc, b_enc_c, cb, w_dec_r, b_dec_r)


def kernel(x, W_enc, b_enc, codebook, W_dec, b_dec):
    T, B, _ = x.shape
    Kc, D = codebook.shape
    n = T * B
    G = n // _R
    x_flat = x.reshape(n, 2)
    out = _run(
        x_flat[:, 0].reshape(G, 1, _R),
        x_flat[:, 1].reshape(G, 1, _R),
        W_enc.T,
        b_enc.reshape(D, 1),
        codebook,
        W_dec.reshape(1, D),
        b_dec.reshape(1, 1),
    )
    return out.reshape(T, B, 1)
